# baseline (speedup 1.0000x reference)
import functools

import jax
import jax.numpy as jnp
from jax import lax
from jax.experimental import pallas as pl
from jax.experimental.pallas import tpu as pltpu
from jax.experimental.pallas import tpu_sc as plsc

IMG_H, IMG_W = 32, 88
BEV_H, BEV_W = 200, 200
X_MIN, X_MAX, Y_MIN, Y_MAX = -50.0, 50.0, -50.0, 50.0
RES_X = (X_MAX - X_MIN) / BEV_W
RES_Y = (Y_MAX - Y_MIN) / BEV_H
C = 256
D = 64
P = IMG_H * IMG_W
G = BEV_H * BEV_W
N = D * P
B = 2

NSC = 2
NTILE = 16
NSLAB = C // 16
PPT = N // NTILE
NWIN = 8
WP = PPT // NWIN
NGRP_W = WP // 16
NCHUNK = PPT // 128
G_PAD = 40960
STRIPE = G_PAD // NTILE
NZCH = STRIPE // 128
SLAB_PER_SC = NSLAB // NSC


def _geometry(depth_values, K, T):
    xs = jnp.linspace(0.0, IMG_W - 1, IMG_W)
    ys = jnp.linspace(0.0, IMG_H - 1, IMG_H)
    gy, gx = jnp.meshgrid(ys, xs, indexing='ij')
    grid = jnp.stack([gx, gy, jnp.ones_like(gx)], axis=-1)
    K_inv = jnp.linalg.inv(K)
    rays = (K_inv @ grid.reshape(-1, 3).T).T.reshape(IMG_H, IMG_W, 3)
    points_cam = depth_values.reshape(-1, 1, 1, 1) * rays[None]
    Rm = T[:3, :3]
    t = T[:3, 3]
    flat = points_cam.reshape(-1, 3)
    points_lidar = ((Rm @ flat.T).T + t).reshape(D, IMG_H, IMG_W, 3)
    x = points_lidar[..., 0]
    y = points_lidar[..., 1]
    z = points_lidar[..., 2]
    bev_x = ((x - X_MIN) / RES_X).astype(jnp.int32)
    bev_y = ((y - Y_MIN) / RES_Y).astype(jnp.int32)
    valid = (bev_x >= 0) & (bev_x < BEV_W) & (bev_y >= 0) & (bev_y < BEV_H) & (z > 0)
    idx = bev_y * BEV_W + bev_x
    return idx.reshape(N), valid.reshape(N)


def _sc_scatter(fslab, idx_h, w_h):
    mesh = plsc.VectorSubcoreMesh(core_axis_name="c", subcore_axis_name="s")

    @functools.partial(
        pl.kernel,
        out_type=jax.ShapeDtypeStruct((B, NSLAB, G_PAD, 16), jnp.float32),
        mesh=mesh,
        compiler_params=pltpu.CompilerParams(
            use_tc_tiling_on_sc=False, needs_layout_passes=False),
        scratch_types=[
            pltpu.VMEM((16, P), jnp.float32),
            pltpu.VMEM((WP,), jnp.int32),
            pltpu.VMEM((WP,), jnp.float32),
            pltpu.VMEM((WP,), jnp.float32),
            pltpu.VMEM((PPT,), jnp.int32),
            pltpu.VMEM((NCHUNK, 128), jnp.int32),
            pltpu.VMEM((PPT,), jnp.float32),
            pltpu.VMEM((128, 16), jnp.float32),
            pltpu.VMEM((16,), jnp.int32),
            pltpu.VMEM((NTILE, 16), jnp.int32),
            pltpu.VMEM_SHARED((G_PAD, 16), jnp.float32),
            pltpu.VMEM_SHARED((NTILE, 16), jnp.int32),
            pltpu.SemaphoreType.DMA,
        ],
    )
    def k(fslab_h, idxf_h, wf_h, out_h,
          f_v, iw_v, w0_v, w1_v, pixc_v, idxc_v, wc_v,
          stage_v, kv_v, kall_v, acc_sh, kt_sh, sem):
        c = lax.axis_index("c")
        s = lax.axis_index("s")
        iota = lax.iota(jnp.int32, 16)
        zi16 = jnp.zeros((16,), jnp.int32)
        zf16 = jnp.zeros((16,), jnp.float32)

        def zero_stage():
            def zrow(i, _):
                stage_v[i, :] = zf16
                return 0
            lax.fori_loop(0, 128, zrow, 0, unroll=8)

        def rezero_stripe():
            descs = []
            for z in range(NZCH):
                descs.append(pltpu.async_copy(
                    stage_v, acc_sh.at[pl.ds(s * STRIPE + z * 128, 128)], sem))
            for d in descs:
                d.wait()

        def prefill(g, _):
            base = g * 16
            pixc_v[pl.ds(base, 16)] = zi16
            wc_v[pl.ds(base, 16)] = zf16
            spread = lax.rem(base + s * PPT + iota, G_PAD)
            idxc_v[base // 128, pl.ds(base % 128, 16)] = spread
            return 0
        lax.fori_loop(0, PPT // 16, prefill, 0, unroll=4)

        def compact(wsel):
            def window(win, off):
                pltpu.sync_copy(idxf_h.at[s, win], iw_v)
                pltpu.sync_copy(wf_h.at[0, s, win], w0_v)
                pltpu.sync_copy(wf_h.at[1, s, win], w1_v)

                def grp(g, off):
                    sl = pl.ds(g * 16, 16)
                    w0 = w0_v[sl]
                    w1 = w1_v[sl]
                    m = (w0 != 0.0) | (w1 != 0.0)
                    mi = jnp.where(m, 1, 0)
                    dest = off + plsc.cumsum(mi) - 1
                    plsc.store_scatter(wc_v, [dest], w1 if wsel else w0, mask=m)
                    if not wsel:
                        pv = lax.rem(win * WP + g * 16 + iota, P)
                        rows = lax.shift_right_logical(dest, 7)
                        cols = lax.bitwise_and(dest, 127)
                        plsc.store_scatter(pixc_v, [dest], pv, mask=m)
                        plsc.store_scatter(idxc_v, [rows, cols], iw_v[sl], mask=m)
                    return off + jnp.sum(mi)
                return lax.fori_loop(0, NGRP_W, grp, off)
            return lax.fori_loop(0, NWIN, window, jnp.int32(0))

        kcnt = compact(0)

        kv_v[...] = jnp.full((16,), kcnt, jnp.int32)
        pltpu.sync_copy(kv_v, kt_sh.at[s])
        zero_stage()
        plsc.subcore_barrier()
        pltpu.sync_copy(kt_sh, kall_v)
        ktot = jnp.sum(plsc.load_gather(kall_v, [iota, zi16]))
        rezero_stripe()
        plsc.subcore_barrier()

        nch = (kcnt + 127) // 128

        def make_sb_body(b):
            def sb_body(i, _):
                slab = NSC * i + c

                @pl.when(nch > 0)
                def _scatter():
                    pltpu.sync_copy(
                        fslab_h.at[b, pl.ds(slab * 16, 16)], f_v)

                    def ch_body(ch, _):
                        for g16 in range(8):
                            sl = pl.ds(ch * 128 + g16 * 16, 16)
                            pv = pixc_v[sl]
                            wv = wc_v[sl]
                            for cc in range(16):
                                ccv = jnp.full((16,), cc, jnp.int32)
                                vals = plsc.load_gather(f_v, [ccv, pv])
                                plsc.store_scatter(
                                    stage_v, [iota + g16 * 16, ccv], vals * wv)
                        pltpu.sync_copy(stage_v, acc_sh.at[idxc_v.at[ch]],
                                        add=True)
                        return 0
                    lax.fori_loop(0, nch, ch_body, 0)

                plsc.subcore_barrier()
                pltpu.sync_copy(
                    acc_sh.at[pl.ds(s * STRIPE, STRIPE)],
                    out_h.at[b, slab, pl.ds(s * STRIPE, STRIPE)])

                @pl.when(ktot > 0)
                def _rezero():
                    zero_stage()
                    rezero_stripe()
                plsc.subcore_barrier()
                return 0
            return sb_body

        for b in range(B):
            if b > 0:
                compact(b)
            lax.fori_loop(0, SLAB_PER_SC, make_sb_body(b), 0)

    return k(fslab, idx_h, w_h)


def _tc_fix_body(in_ref, out_ref):
    x = in_ref[0, 0]
    xt = jnp.transpose(x, (1, 0))
    for j in range(8):
        out_ref[0, :, j, :] = xt[j * 16:(j + 1) * 16, :]


def _tc_transpose(out_sc):
    return pl.pallas_call(
        _tc_fix_body,
        grid=(B, NSLAB, BEV_H // 8),
        in_specs=[pl.BlockSpec((1, 1, 200, 128), lambda b, sl, t: (b, sl, t, 0))],
        out_specs=pl.BlockSpec((1, 16, 8, 200), lambda b, sl, t: (b, sl, t, 0)),
        out_shape=jax.ShapeDtypeStruct((B, C, BEV_H, BEV_W), jnp.float32),
    )(out_sc.reshape(B, NSLAB, G_PAD // 8, 128))


def kernel(img_features, depth_probs, depth_values, K, T):
    idx, valid = _geometry(depth_values, K, T)
    narange = jnp.arange(N, dtype=jnp.int32)
    t_blk = idx // 1600
    u = idx % 1600
    row = t_blk * 1600 + (u % 200) * 8 + u // 200
    idx_sp = jnp.where(valid, row, narange % G_PAD)
    w = (depth_probs.reshape(B, N) *
         valid.astype(jnp.float32)[None])

    out = _sc_scatter(
        img_features.reshape(B, C, P),
        idx_sp.reshape(NTILE, NWIN, WP),
        w.reshape(B, NTILE, NWIN, WP),
    )
    return _tc_transpose(out)


# speedup vs baseline: 1.8080x; 1.8080x over previous
import functools

import jax
import jax.numpy as jnp
from jax import lax
from jax.experimental import pallas as pl
from jax.experimental.pallas import tpu as pltpu
from jax.experimental.pallas import tpu_sc as plsc

IMG_H, IMG_W = 32, 88
BEV_H, BEV_W = 200, 200
X_MIN, X_MAX, Y_MIN, Y_MAX = -50.0, 50.0, -50.0, 50.0
RES_X = (X_MAX - X_MIN) / BEV_W
RES_Y = (Y_MAX - Y_MIN) / BEV_H
C = 256
D = 64
P = IMG_H * IMG_W
G = BEV_H * BEV_W
N = D * P
B = 2

NSC = 2
NTILE = 16
NSLAB = C // 16
PPT = N // NTILE
NWIN = 8
WP = PPT // NWIN
NGRP_W = WP // 16
NCHUNK = PPT // 128
G_PAD = 40960
STRIPE = G_PAD // NTILE
NZCH = STRIPE // 128
SLAB_PER_SC = NSLAB // NSC


def _geometry(depth_values, K, T):
    xs = jnp.linspace(0.0, IMG_W - 1, IMG_W)
    ys = jnp.linspace(0.0, IMG_H - 1, IMG_H)
    gy, gx = jnp.meshgrid(ys, xs, indexing='ij')
    grid = jnp.stack([gx, gy, jnp.ones_like(gx)], axis=-1)
    K_inv = jnp.linalg.inv(K)
    rays = (K_inv @ grid.reshape(-1, 3).T).T.reshape(IMG_H, IMG_W, 3)
    points_cam = depth_values.reshape(-1, 1, 1, 1) * rays[None]
    Rm = T[:3, :3]
    t = T[:3, 3]
    flat = points_cam.reshape(-1, 3)
    points_lidar = ((Rm @ flat.T).T + t).reshape(D, IMG_H, IMG_W, 3)
    x = points_lidar[..., 0]
    y = points_lidar[..., 1]
    z = points_lidar[..., 2]
    bev_x = ((x - X_MIN) / RES_X).astype(jnp.int32)
    bev_y = ((y - Y_MIN) / RES_Y).astype(jnp.int32)
    valid = (bev_x >= 0) & (bev_x < BEV_W) & (bev_y >= 0) & (bev_y < BEV_H) & (z > 0)
    idx = bev_y * BEV_W + bev_x
    return idx.reshape(N), valid.reshape(N)


def _sc_scatter(fslab, idx_h, w_h):
    mesh = plsc.VectorSubcoreMesh(core_axis_name="c", subcore_axis_name="s")

    @functools.partial(
        pl.kernel,
        out_type=jax.ShapeDtypeStruct((B, NSLAB, G_PAD, 16), jnp.float32),
        mesh=mesh,
        compiler_params=pltpu.CompilerParams(
            use_tc_tiling_on_sc=False, needs_layout_passes=False),
        scratch_types=[
            pltpu.VMEM((16, P), jnp.float32),
            pltpu.VMEM((WP,), jnp.int32),
            pltpu.VMEM((WP,), jnp.float32),
            pltpu.VMEM((WP,), jnp.float32),
            pltpu.VMEM((PPT,), jnp.int32),
            pltpu.VMEM((NCHUNK, 128), jnp.int32),
            pltpu.VMEM((PPT,), jnp.float32),
            pltpu.VMEM((128, 16), jnp.float32),
            pltpu.VMEM((16,), jnp.int32),
            pltpu.VMEM((NTILE, 16), jnp.int32),
            pltpu.VMEM_SHARED((G_PAD, 16), jnp.float32),
            pltpu.VMEM_SHARED((NTILE, 16), jnp.int32),
            pltpu.SemaphoreType.DMA,
        ],
    )
    def k(fslab_h, idxf_h, wf_h, out_h,
          f_v, iw_v, w0_v, w1_v, pixc_v, idxc_v, wc_v,
          stage_v, kv_v, kall_v, acc_sh, kt_sh, sem):
        c = lax.axis_index("c")
        s = lax.axis_index("s")
        iota = lax.iota(jnp.int32, 16)
        zi16 = jnp.zeros((16,), jnp.int32)
        zf16 = jnp.zeros((16,), jnp.float32)

        def zero_stage():
            def zrow(i, _):
                stage_v[i, :] = zf16
                return 0
            lax.fori_loop(0, 128, zrow, 0, unroll=8)

        def rezero_stripe():
            descs = []
            for z in range(NZCH):
                descs.append(pltpu.async_copy(
                    stage_v, acc_sh.at[pl.ds(s * STRIPE + z * 128, 128)], sem))
            for d in descs:
                d.wait()

        def prefill(g, _):
            base = g * 16
            pixc_v[pl.ds(base, 16)] = zi16
            wc_v[pl.ds(base, 16)] = zf16
            spread = lax.rem(base + s * PPT + iota, G_PAD)
            idxc_v[base // 128, pl.ds(base % 128, 16)] = spread
            return 0
        lax.fori_loop(0, PPT // 16, prefill, 0, unroll=4)

        def compact(wsel):
            def window(win, off):
                pltpu.sync_copy(idxf_h.at[s, win], iw_v)
                pltpu.sync_copy(wf_h.at[0, s, win], w0_v)
                pltpu.sync_copy(wf_h.at[1, s, win], w1_v)

                def grp(g, off):
                    sl = pl.ds(g * 16, 16)
                    w0 = w0_v[sl]
                    w1 = w1_v[sl]
                    m = (w0 != 0.0) | (w1 != 0.0)
                    mi = jnp.where(m, 1, 0)
                    dest = off + plsc.cumsum(mi) - 1
                    plsc.store_scatter(wc_v, [dest], w1 if wsel else w0, mask=m)
                    if not wsel:
                        pv = lax.rem(win * WP + g * 16 + iota, P)
                        rows = lax.shift_right_logical(dest, 7)
                        cols = lax.bitwise_and(dest, 127)
                        plsc.store_scatter(pixc_v, [dest], pv, mask=m)
                        plsc.store_scatter(idxc_v, [rows, cols], iw_v[sl], mask=m)
                    return off + jnp.sum(mi)
                return lax.fori_loop(0, NGRP_W, grp, off)
            return lax.fori_loop(0, NWIN, window, jnp.int32(0))

        kcnt = compact(0)

        kv_v[...] = jnp.full((16,), kcnt, jnp.int32)
        pltpu.sync_copy(kv_v, kt_sh.at[s])
        zero_stage()
        plsc.subcore_barrier()
        pltpu.sync_copy(kt_sh, kall_v)
        ktot = jnp.sum(plsc.load_gather(kall_v, [iota, zi16]))
        rezero_stripe()
        plsc.subcore_barrier()

        nch = (kcnt + 127) // 128

        def make_sb_body(b):
            def sb_body(i, _):
                slab = NSC * i + c

                @pl.when(nch > 0)
                def _scatter():
                    pltpu.sync_copy(
                        fslab_h.at[b, pl.ds(slab * 16, 16)], f_v)

                    def ch_body(ch, _):
                        for g16 in range(8):
                            sl = pl.ds(ch * 128 + g16 * 16, 16)
                            pv = pixc_v[sl]
                            wv = wc_v[sl]
                            for cc in range(16):
                                ccv = jnp.full((16,), cc, jnp.int32)
                                vals = plsc.load_gather(f_v, [ccv, pv])
                                plsc.store_scatter(
                                    stage_v, [iota + g16 * 16, ccv], vals * wv)
                        pltpu.sync_copy(stage_v, acc_sh.at[idxc_v.at[ch]],
                                        add=True)
                        return 0
                    lax.fori_loop(0, nch, ch_body, 0)

                plsc.subcore_barrier()
                pltpu.sync_copy(
                    acc_sh.at[pl.ds(s * STRIPE, STRIPE)],
                    out_h.at[b, slab, pl.ds(s * STRIPE, STRIPE)])

                @pl.when(ktot > 0)
                def _rezero():
                    zero_stage()
                    rezero_stripe()
                plsc.subcore_barrier()
                return 0
            return sb_body

        for b in range(B):
            if b > 0:
                compact(b)
            lax.fori_loop(0, SLAB_PER_SC, make_sb_body(b), 0)

    return k(fslab, idx_h, w_h)


def _tc_fix_body(in_ref, out_ref):
    for tt in range(5):
        x = in_ref[0, 0, tt * 200:(tt + 1) * 200, :]
        xt = jnp.transpose(x, (1, 0))
        for j in range(8):
            out_ref[0, :, tt * 8 + j, :] = xt[j * 16:(j + 1) * 16, :]


def _tc_transpose(out_sc):
    return pl.pallas_call(
        _tc_fix_body,
        grid=(B, NSLAB, BEV_H // 40),
        in_specs=[pl.BlockSpec((1, 1, 1000, 128), lambda b, sl, t: (b, sl, t, 0))],
        out_specs=pl.BlockSpec((1, 16, 40, 200), lambda b, sl, t: (b, sl, t, 0)),
        out_shape=jax.ShapeDtypeStruct((B, C, BEV_H, BEV_W), jnp.float32),
    )(out_sc.reshape(B, NSLAB, G_PAD // 8, 128))


def kernel(img_features, depth_probs, depth_values, K, T):
    idx, valid = _geometry(depth_values, K, T)
    narange = jnp.arange(N, dtype=jnp.int32)
    t_blk = idx // 1600
    u = idx % 1600
    row = t_blk * 1600 + (u % 200) * 8 + u // 200
    idx_sp = jnp.where(valid, row, narange % G_PAD)
    w = (depth_probs.reshape(B, N) *
         valid.astype(jnp.float32)[None])

    out = _sc_scatter(
        img_features.reshape(B, C, P),
        idx_sp.reshape(NTILE, NWIN, WP),
        w.reshape(B, NTILE, NWIN, WP),
    )
    return _tc_transpose(out)
